# baseline (speedup 1.0000x reference)
import functools

import jax
import jax.numpy as jnp
from jax import lax
from jax.experimental import pallas as pl
from jax.experimental.pallas import tpu as pltpu
from jax.experimental.pallas import tpu_sc as plsc

NC = 2
NS = 16
NW = NC * NS
CHUNK = 128
LANES = 16


def _i32(v):
    return jnp.int32(v)


def _prep_body(x_ref, wn_ref, bn_ref, wm1a_ref, xt_ref, p_ref):
    xt = jnp.dot(x_ref[...], wn_ref[...], preferred_element_type=jnp.float32)
    xt = xt + bn_ref[...]
    xt_ref[...] = xt
    p_ref[...] = jnp.dot(xt, wm1a_ref[...], preferred_element_type=jnp.float32)


def _tc_prep(x, wn, bn2, wm1a, bn_blk):
    n, d = x.shape
    h = wn.shape[1]
    grid = (n // bn_blk,)
    return pl.pallas_call(
        _prep_body,
        grid=grid,
        in_specs=[
            pl.BlockSpec((bn_blk, d), lambda i: (i, i * 0)),
            pl.BlockSpec((d, h), lambda i: (i * 0, i * 0)),
            pl.BlockSpec((1, h), lambda i: (i * 0, i * 0)),
            pl.BlockSpec((h, h), lambda i: (i * 0, i * 0)),
        ],
        out_specs=[
            pl.BlockSpec((bn_blk, h), lambda i: (i, i * 0)),
            pl.BlockSpec((bn_blk, h), lambda i: (i, i * 0)),
        ],
        out_shape=[
            jax.ShapeDtypeStruct((n, h), jnp.float32),
            jax.ShapeDtypeStruct((n, h), jnp.float32),
        ],
    )(x, wn, bn2, wm1a)


def _qmat_body(ea_ref, we_ref, be_ref, wm1b_ref, bm1_ref, q_ref):
    w2 = jnp.dot(we_ref[...], wm1b_ref[...], preferred_element_type=jnp.float32)
    c = jnp.dot(be_ref[...], wm1b_ref[...], preferred_element_type=jnp.float32)
    c = c + bm1_ref[...]
    q_ref[...] = jnp.dot(ea_ref[...], w2, preferred_element_type=jnp.float32) + c


def _tc_qmat(ea_p, we, be2, wm1b, bm12, be_blk):
    ep, ed = ea_p.shape
    h = wm1b.shape[1]
    grid = (ep // be_blk,)
    return pl.pallas_call(
        _qmat_body,
        grid=grid,
        in_specs=[
            pl.BlockSpec((be_blk, ed), lambda i: (i, i * 0)),
            pl.BlockSpec((ed, h), lambda i: (i * 0, i * 0)),
            pl.BlockSpec((1, h), lambda i: (i * 0, i * 0)),
            pl.BlockSpec((h, h), lambda i: (i * 0, i * 0)),
            pl.BlockSpec((1, h), lambda i: (i * 0, i * 0)),
        ],
        out_specs=pl.BlockSpec((be_blk, h), lambda i: (i, i * 0)),
        out_shape=jax.ShapeDtypeStruct((ep, h), jnp.float32),
    )(ea_p, we, be2, wm1b, bm12)


def _post_body(s_ref, cnt_ref, p_ref, xt_ref, xp_ref, we_ref, be_ref,
               wm1b_ref, bm1_ref, wm2_ref, bm2_ref, wu1a_ref, wu1b_ref,
               bu1_ref, wu2_ref, bu2_ref, g_ref, b_ref, rs_ref, out_ref):
    f32 = jnp.float32
    c = jnp.dot(be_ref[...], wm1b_ref[...], preferred_element_type=f32)
    c = c + bm1_ref[...]
    qself = jnp.dot(jnp.sum(we_ref[...], axis=0, keepdims=True), wm1b_ref[...],
                    preferred_element_type=f32) + c
    s = s_ref[...]
    big_s = s[0] + s[1] + jnp.maximum(p_ref[...] + qself, 0.0)
    cnt = cnt_ref[...]
    cnt = jnp.maximum(cnt[0, :, 0:1] + cnt[1, :, 0:1] + 1.0, 1.0)
    aggr = jnp.dot(big_s, wm2_ref[...], preferred_element_type=f32) / cnt
    aggr = aggr + bm2_ref[...]
    h2 = jnp.dot(aggr, wu1a_ref[...], preferred_element_type=f32)
    h2 = h2 + jnp.dot(xt_ref[...], wu1b_ref[...], preferred_element_type=f32)
    h2 = jnp.maximum(h2 + bu1_ref[...], 0.0)
    o = jnp.dot(h2, wu2_ref[...], preferred_element_type=f32) + bu2_ref[...]
    mu = jnp.mean(o, axis=-1, keepdims=True)
    var = jnp.mean((o - mu) ** 2, axis=-1, keepdims=True)
    ln = (o - mu) / jnp.sqrt(var + 1e-5) * g_ref[...] + b_ref[...]
    rs = jnp.maximum(rs_ref[0, 0], 0.0)
    out_ref[...] = ln + rs * xp_ref[...]


def _tc_post(s2, cnt2, p, xt, xp, we, be2, wm1b, bm12, wm2, bm22,
             wu1a, wu1b, bu12, wu2, bu22, g2, b2, rs, bn_blk):
    n, h = p.shape
    ed = we.shape[0]
    cw = cnt2.shape[2]
    grid = (n // bn_blk,)
    full = lambda r, cdim: pl.BlockSpec((r, cdim), lambda i: (i * 0, i * 0))
    return pl.pallas_call(
        _post_body,
        grid=grid,
        in_specs=[
            pl.BlockSpec((2, bn_blk, h), lambda i: (i * 0, i, i * 0)),
            pl.BlockSpec((2, bn_blk, cw), lambda i: (i * 0, i, i * 0)),
            pl.BlockSpec((bn_blk, h), lambda i: (i, i * 0)),
            pl.BlockSpec((bn_blk, h), lambda i: (i, i * 0)),
            pl.BlockSpec((bn_blk, h), lambda i: (i, i * 0)),
            full(ed, h), full(1, h), full(h, h), full(1, h),
            full(h, h), full(1, h), full(h, h), full(h, h), full(1, h),
            full(h, h), full(1, h), full(1, h), full(1, h), full(1, 1),
        ],
        out_specs=pl.BlockSpec((bn_blk, h), lambda i: (i, i * 0)),
        out_shape=jax.ShapeDtypeStruct((n, h), jnp.float32),
    )(s2, cnt2, p, xt, xp, we, be2, wm1b, bm12, wm2, bm22,
      wu1a, wu1b, bu12, wu2, bu22, g2, b2, rs)


def _make_edge_sc(n_pad, h, epw, nchunks, ec):
    mesh = plsc.VectorSubcoreMesh(core_axis_name="c", subcore_axis_name="s")
    rows_per_tile = n_pad // NS
    assert nchunks % 3 == 0 and nchunks >= 6
    out_type = [jax.ShapeDtypeStruct((NC, n_pad, h), jnp.float32)]
    scratch = (
        [pltpu.VMEM_SHARED((n_pad, h), jnp.float32)]
        + [pltpu.VMEM((ec,), jnp.int32) for _ in range(3)]
        + [pltpu.VMEM((ec,), jnp.int32) for _ in range(3)]
        + [pltpu.VMEM((ec, h), jnp.float32) for _ in range(3)]
        + [pltpu.VMEM((ec, h), jnp.float32) for _ in range(3)]
        + [pltpu.SemaphoreType.DMA for _ in range(9)]
    )

    def body(p_hbm, q_hbm, src_hbm, dst_hbm, out_s, s_sh, *rest):
        srcv = rest[0:3]
        dstv = rest[3:6]
        rows = rest[6:9]
        qv = rest[9:12]
        gsem = rest[12:15]
        qsem = rest[15:18]
        ssem = rest[18:21]
        cid = lax.axis_index("c").astype(jnp.int32)
        sid = lax.axis_index("s").astype(jnp.int32)
        wid = cid * _i32(NS) + sid
        row0 = sid * _i32(rows_per_tile)
        base = wid * _i32(epw)

        @pl.loop(_i32(0), _i32(ec))
        def _zero_q(r):
            for cc in range(h // LANES):
                qv[0][r, pl.ds(cc * LANES, LANES)] = jnp.zeros((LANES,), jnp.float32)

        left = rows_per_tile
        while left > 0:
            step = min(ec, left)
            pltpu.sync_copy(qv[0].at[pl.ds(0, step)],
                            s_sh.at[pl.ds(row0 + _i32(rows_per_tile - left), step)])
            left -= step

        plsc.subcore_barrier()

        def load(b, g):
            off = pl.multiple_of(base + g * _i32(ec), ec)
            pltpu.sync_copy(src_hbm.at[pl.ds(off, ec)], srcv[b])
            pltpu.sync_copy(dst_hbm.at[pl.ds(off, ec)], dstv[b])
            pltpu.async_copy(p_hbm.at[srcv[b]], rows[b], gsem[b])
            pltpu.async_copy(q_hbm.at[pl.ds(off, ec)], qv[b], qsem[b])

        def process(b):
            pltpu.make_async_copy(p_hbm.at[srcv[b]], rows[b], gsem[b]).wait()
            pltpu.make_async_copy(q_hbm.at[pl.ds(0, ec)], qv[b], qsem[b]).wait()

            @pl.loop(_i32(0), _i32(ec))
            def _relu(r):
                for cc in range(h // LANES):
                    sl = pl.ds(cc * LANES, LANES)
                    qv[b][r, sl] = jnp.maximum(rows[b][r, sl] + qv[b][r, sl], 0.0)

            pltpu.async_copy(qv[b], s_sh.at[dstv[b]], ssem[b], add=True)

        def drain_scatter(b):
            pltpu.make_async_copy(q_hbm.at[pl.ds(0, ec)], qv[b], ssem[b]).wait()

        for b in range(3):
            load(b, _i32(b))

        @pl.loop(_i32(0), _i32(nchunks - 3), step=3)
        def _outer(g0):
            for b in range(3):
                process(b)
            for b in range(3):
                drain_scatter(b)
                load(b, g0 + _i32(b + 3))

        for b in range(3):
            process(b)
        for b in range(3):
            drain_scatter(b)

        plsc.subcore_barrier()
        pltpu.sync_copy(s_sh.at[pl.ds(row0, rows_per_tile)],
                        out_s.at[cid, pl.ds(row0, rows_per_tile)])

    return pl.kernel(body, out_type=out_type, mesh=mesh, scratch_types=scratch)


def _make_cnt_sc(n_pad, epw, nchunks, cw):
    mesh = plsc.VectorSubcoreMesh(core_axis_name="c", subcore_axis_name="s")
    rows_per_tile = n_pad // NS
    out_type = [jax.ShapeDtypeStruct((NC, n_pad, cw), jnp.float32)]
    scratch = [
        pltpu.VMEM_SHARED((n_pad, cw), jnp.float32),
        pltpu.VMEM((CHUNK,), jnp.int32),
        pltpu.VMEM((CHUNK, cw), jnp.float32),
        pltpu.SemaphoreType.DMA,
    ]

    def body(dst_hbm, out_c, c_sh, dst_v, ones_v, sem):
        cid = lax.axis_index("c").astype(jnp.int32)
        sid = lax.axis_index("s").astype(jnp.int32)
        wid = cid * _i32(NS) + sid
        row0 = sid * _i32(rows_per_tile)

        @pl.loop(_i32(0), _i32(CHUNK))
        def _zero(r):
            for cc in range(cw // LANES):
                ones_v[r, pl.ds(cc * LANES, LANES)] = jnp.zeros((LANES,), jnp.float32)

        left = rows_per_tile
        while left > 0:
            step = min(CHUNK, left)
            pltpu.sync_copy(ones_v.at[pl.ds(0, step)],
                            c_sh.at[pl.ds(row0 + _i32(rows_per_tile - left), step)])
            left -= step

        @pl.loop(_i32(0), _i32(CHUNK))
        def _ones(r):
            for cc in range(cw // LANES):
                ones_v[r, pl.ds(cc * LANES, LANES)] = jnp.ones((LANES,), jnp.float32)

        plsc.subcore_barrier()

        base = wid * _i32(epw)

        @pl.loop(_i32(0), _i32(nchunks))
        def _edges(g):
            off = pl.multiple_of(base + g * _i32(CHUNK), CHUNK)
            pltpu.sync_copy(dst_hbm.at[pl.ds(off, CHUNK)], dst_v)
            pltpu.sync_copy(ones_v, c_sh.at[dst_v], add=True)

        plsc.subcore_barrier()
        pltpu.sync_copy(c_sh.at[pl.ds(row0, rows_per_tile)],
                        out_c.at[cid, pl.ds(row0, rows_per_tile)])

    return pl.kernel(body, out_type=out_type, mesh=mesh, scratch_types=scratch)


def kernel(x, edge_index, edge_attr, params):
    n, d = x.shape
    e = edge_index.shape[1]
    ed = edge_attr.shape[1]
    layers = params['layers']
    h = layers[0]['Wn'].shape[1]
    cw = h

    ec = 64
    gran = 384
    epw = -(-e // (NW * gran)) * gran
    ep = epw * NW
    n_pad = -(-(n + 1) // CHUNK) * CHUNK
    bn_blk = 2000 if n % 2000 == 0 else 1000
    be_blk = 4096 if ep % 4096 == 0 else CHUNK

    src = edge_index[0].astype(jnp.int32)
    dst = edge_index[1].astype(jnp.int32)
    pad = ep - e
    src_p = jnp.concatenate([src, jnp.zeros((pad,), jnp.int32)])
    dst_p = jnp.concatenate([dst, jnp.full((pad,), n, jnp.int32)])
    ea_p = jnp.concatenate(
        [edge_attr.astype(jnp.float32),
         jnp.zeros((pad, ed), jnp.float32)], axis=0)

    edge_sc = _make_edge_sc(n_pad, h, epw, epw // ec, ec)
    cnt_sc = _make_cnt_sc(n_pad, epw, epw // CHUNK, cw)

    (cnt2,) = cnt_sc(dst_p)

    xin = x.astype(jnp.float32)
    xprev = None
    skip = params['skip'].astype(jnp.float32)
    for i, p in enumerate(layers):
        wm1a = p['Wm1'][:h]
        wm1b = p['Wm1'][h:]
        xt, pmat = _tc_prep(xin, p['Wn'], p['bn'][None], wm1a, bn_blk)
        q = _tc_qmat(ea_p, p['We'], p['be'][None], wm1b, p['bm1'][None], be_blk)
        (s2,) = edge_sc(pmat, q, src_p, dst_p)
        if i == 0:
            rs = jnp.zeros((1, 1), jnp.float32)
            xp = xt
        else:
            rs = skip[i - 1].reshape(1, 1).astype(jnp.float32)
            xp = xprev
        out = _tc_post(s2, cnt2, pmat, xt, xp, p['We'], p['be'][None],
                       wm1b, p['bm1'][None], p['Wm2'], p['bm2'][None],
                       p['Wu1'][:h], p['Wu1'][h:], p['bu1'][None],
                       p['Wu2'], p['bu2'][None], p['ln_g'][None],
                       p['ln_b'][None], rs, bn_blk)
        xprev = out
        xin = out
    return xprev


# speedup vs baseline: 1.1225x; 1.1225x over previous
import functools

import jax
import jax.numpy as jnp
from jax import lax
from jax.experimental import pallas as pl
from jax.experimental.pallas import tpu as pltpu
from jax.experimental.pallas import tpu_sc as plsc

NC = 2
NS = 16
NW = NC * NS
CHUNK = 128
LANES = 16


def _i32(v):
    return jnp.int32(v)


def _prep_body(x_ref, wn_ref, bn_ref, wm1a_ref, xt_ref, p_ref):
    xt = jnp.dot(x_ref[...], wn_ref[...], preferred_element_type=jnp.float32)
    xt = xt + bn_ref[...]
    xt_ref[...] = xt
    p_ref[...] = jnp.dot(xt, wm1a_ref[...], preferred_element_type=jnp.float32)


def _tc_prep(x, wn, bn2, wm1a, bn_blk):
    n, d = x.shape
    h = wn.shape[1]
    grid = (n // bn_blk,)
    return pl.pallas_call(
        _prep_body,
        grid=grid,
        in_specs=[
            pl.BlockSpec((bn_blk, d), lambda i: (i, i * 0)),
            pl.BlockSpec((d, h), lambda i: (i * 0, i * 0)),
            pl.BlockSpec((1, h), lambda i: (i * 0, i * 0)),
            pl.BlockSpec((h, h), lambda i: (i * 0, i * 0)),
        ],
        out_specs=[
            pl.BlockSpec((bn_blk, h), lambda i: (i, i * 0)),
            pl.BlockSpec((bn_blk, h), lambda i: (i, i * 0)),
        ],
        out_shape=[
            jax.ShapeDtypeStruct((n, h), jnp.float32),
            jax.ShapeDtypeStruct((n, h), jnp.float32),
        ],
    )(x, wn, bn2, wm1a)


def _qmat_body(ea_ref, we_ref, be_ref, wm1b_ref, bm1_ref, q_ref):
    w2 = jnp.dot(we_ref[...], wm1b_ref[...], preferred_element_type=jnp.float32)
    c = jnp.dot(be_ref[...], wm1b_ref[...], preferred_element_type=jnp.float32)
    c = c + bm1_ref[...]
    q_ref[...] = jnp.dot(ea_ref[...], w2, preferred_element_type=jnp.float32) + c


def _tc_qmat(ea, ep, we, be2, wm1b, bm12, be_blk):
    e, ed = ea.shape
    h = wm1b.shape[1]
    grid = (e // be_blk,)
    return pl.pallas_call(
        _qmat_body,
        grid=grid,
        in_specs=[
            pl.BlockSpec((be_blk, ed), lambda i: (i, i * 0)),
            pl.BlockSpec((ed, h), lambda i: (i * 0, i * 0)),
            pl.BlockSpec((1, h), lambda i: (i * 0, i * 0)),
            pl.BlockSpec((h, h), lambda i: (i * 0, i * 0)),
            pl.BlockSpec((1, h), lambda i: (i * 0, i * 0)),
        ],
        out_specs=pl.BlockSpec((be_blk, h), lambda i: (i, i * 0)),
        out_shape=jax.ShapeDtypeStruct((ep, h), jnp.float32),
    )(ea, we, be2, wm1b, bm12)


def _post_body(s_ref, cnt_ref, p_ref, xt_ref, xp_ref, we_ref, be_ref,
               wm1b_ref, bm1_ref, wm2_ref, bm2_ref, wu1a_ref, wu1b_ref,
               bu1_ref, wu2_ref, bu2_ref, g_ref, b_ref, rs_ref, out_ref):
    f32 = jnp.float32
    c = jnp.dot(be_ref[...], wm1b_ref[...], preferred_element_type=f32)
    c = c + bm1_ref[...]
    qself = jnp.dot(jnp.sum(we_ref[...], axis=0, keepdims=True), wm1b_ref[...],
                    preferred_element_type=f32) + c
    s = s_ref[...]
    big_s = s[0] + s[1] + jnp.maximum(p_ref[...] + qself, 0.0)
    cnt = cnt_ref[...]
    cnt = jnp.maximum(cnt[0, :, 0:1] + cnt[1, :, 0:1] + 1.0, 1.0)
    aggr = jnp.dot(big_s, wm2_ref[...], preferred_element_type=f32) / cnt
    aggr = aggr + bm2_ref[...]
    h2 = jnp.dot(aggr, wu1a_ref[...], preferred_element_type=f32)
    h2 = h2 + jnp.dot(xt_ref[...], wu1b_ref[...], preferred_element_type=f32)
    h2 = jnp.maximum(h2 + bu1_ref[...], 0.0)
    o = jnp.dot(h2, wu2_ref[...], preferred_element_type=f32) + bu2_ref[...]
    mu = jnp.mean(o, axis=-1, keepdims=True)
    var = jnp.mean((o - mu) ** 2, axis=-1, keepdims=True)
    ln = (o - mu) / jnp.sqrt(var + 1e-5) * g_ref[...] + b_ref[...]
    rs = jnp.maximum(rs_ref[0, 0], 0.0)
    out_ref[...] = ln + rs * xp_ref[...]


def _tc_post(s2, cnt2, p, xt, xp, we, be2, wm1b, bm12, wm2, bm22,
             wu1a, wu1b, bu12, wu2, bu22, g2, b2, rs, bn_blk):
    n, h = p.shape
    ed = we.shape[0]
    cw = cnt2.shape[2]
    grid = (n // bn_blk,)
    full = lambda r, cdim: pl.BlockSpec((r, cdim), lambda i: (i * 0, i * 0))
    return pl.pallas_call(
        _post_body,
        grid=grid,
        in_specs=[
            pl.BlockSpec((2, bn_blk, h), lambda i: (i * 0, i, i * 0)),
            pl.BlockSpec((2, bn_blk, cw), lambda i: (i * 0, i, i * 0)),
            pl.BlockSpec((bn_blk, h), lambda i: (i, i * 0)),
            pl.BlockSpec((bn_blk, h), lambda i: (i, i * 0)),
            pl.BlockSpec((bn_blk, h), lambda i: (i, i * 0)),
            full(ed, h), full(1, h), full(h, h), full(1, h),
            full(h, h), full(1, h), full(h, h), full(h, h), full(1, h),
            full(h, h), full(1, h), full(1, h), full(1, h), full(1, 1),
        ],
        out_specs=pl.BlockSpec((bn_blk, h), lambda i: (i, i * 0)),
        out_shape=jax.ShapeDtypeStruct((n, h), jnp.float32),
    )(s2, cnt2, p, xt, xp, we, be2, wm1b, bm12, wm2, bm22,
      wu1a, wu1b, bu12, wu2, bu22, g2, b2, rs)


def _make_edge_sc(n_pad, h, epw, nchunks, ec):
    mesh = plsc.VectorSubcoreMesh(core_axis_name="c", subcore_axis_name="s")
    rows_per_tile = n_pad // NS
    assert nchunks % 3 == 0 and nchunks >= 6
    out_type = [jax.ShapeDtypeStruct((NC, n_pad, h), jnp.float32)]
    scratch = (
        [pltpu.VMEM_SHARED((n_pad, h), jnp.float32)]
        + [pltpu.VMEM((ec,), jnp.int32) for _ in range(3)]
        + [pltpu.VMEM((ec,), jnp.int32) for _ in range(3)]
        + [pltpu.VMEM((ec, h), jnp.float32) for _ in range(3)]
        + [pltpu.VMEM((ec, h), jnp.float32) for _ in range(3)]
        + [pltpu.SemaphoreType.DMA for _ in range(9)]
    )

    def body(p_hbm, q_hbm, src_hbm, dst_hbm, out_s, s_sh, *rest):
        srcv = rest[0:3]
        dstv = rest[3:6]
        rows = rest[6:9]
        qv = rest[9:12]
        gsem = rest[12:15]
        qsem = rest[15:18]
        ssem = rest[18:21]
        cid = lax.axis_index("c").astype(jnp.int32)
        sid = lax.axis_index("s").astype(jnp.int32)
        wid = cid * _i32(NS) + sid
        row0 = sid * _i32(rows_per_tile)
        base = wid * _i32(epw)

        @pl.loop(_i32(0), _i32(ec))
        def _zero_q(r):
            for cc in range(h // LANES):
                qv[0][r, pl.ds(cc * LANES, LANES)] = jnp.zeros((LANES,), jnp.float32)

        left = rows_per_tile
        while left > 0:
            step = min(ec, left)
            pltpu.sync_copy(qv[0].at[pl.ds(0, step)],
                            s_sh.at[pl.ds(row0 + _i32(rows_per_tile - left), step)])
            left -= step

        plsc.subcore_barrier()

        def load(b, g):
            off = pl.multiple_of(base + g * _i32(ec), ec)
            pltpu.sync_copy(src_hbm.at[pl.ds(off, ec)], srcv[b])
            pltpu.sync_copy(dst_hbm.at[pl.ds(off, ec)], dstv[b])
            pltpu.async_copy(p_hbm.at[srcv[b]], rows[b], gsem[b])
            pltpu.async_copy(q_hbm.at[pl.ds(off, ec)], qv[b], qsem[b])

        def process(b):
            pltpu.make_async_copy(p_hbm.at[srcv[b]], rows[b], gsem[b]).wait()
            pltpu.make_async_copy(q_hbm.at[pl.ds(0, ec)], qv[b], qsem[b]).wait()

            @pl.loop(_i32(0), _i32(ec))
            def _relu(r):
                for cc in range(h // LANES):
                    sl = pl.ds(cc * LANES, LANES)
                    qv[b][r, sl] = jnp.maximum(rows[b][r, sl] + qv[b][r, sl], 0.0)

            pltpu.async_copy(qv[b], s_sh.at[dstv[b]], ssem[b], add=True)

        def drain_scatter(b):
            pltpu.make_async_copy(q_hbm.at[pl.ds(0, ec)], qv[b], ssem[b]).wait()

        for b in range(3):
            load(b, _i32(b))

        @pl.loop(_i32(0), _i32(nchunks - 3), step=3)
        def _outer(g0):
            for b in range(3):
                process(b)
            for b in range(3):
                drain_scatter(b)
                load(b, g0 + _i32(b + 3))

        for b in range(3):
            process(b)
        for b in range(3):
            drain_scatter(b)

        plsc.subcore_barrier()
        pltpu.sync_copy(s_sh.at[pl.ds(row0, rows_per_tile)],
                        out_s.at[cid, pl.ds(row0, rows_per_tile)])

    return pl.kernel(body, out_type=out_type, mesh=mesh, scratch_types=scratch)


def _make_cnt_sc(n_pad, epw, nchunks, cw):
    mesh = plsc.VectorSubcoreMesh(core_axis_name="c", subcore_axis_name="s")
    rows_per_tile = n_pad // NS
    out_type = [jax.ShapeDtypeStruct((NC, n_pad, cw), jnp.float32)]
    scratch = [
        pltpu.VMEM_SHARED((n_pad, cw), jnp.float32),
        pltpu.VMEM((CHUNK,), jnp.int32),
        pltpu.VMEM((CHUNK, cw), jnp.float32),
        pltpu.SemaphoreType.DMA,
    ]

    def body(dst_hbm, out_c, c_sh, dst_v, ones_v, sem):
        cid = lax.axis_index("c").astype(jnp.int32)
        sid = lax.axis_index("s").astype(jnp.int32)
        wid = cid * _i32(NS) + sid
        row0 = sid * _i32(rows_per_tile)

        @pl.loop(_i32(0), _i32(CHUNK))
        def _zero(r):
            for cc in range(cw // LANES):
                ones_v[r, pl.ds(cc * LANES, LANES)] = jnp.zeros((LANES,), jnp.float32)

        left = rows_per_tile
        while left > 0:
            step = min(CHUNK, left)
            pltpu.sync_copy(ones_v.at[pl.ds(0, step)],
                            c_sh.at[pl.ds(row0 + _i32(rows_per_tile - left), step)])
            left -= step

        @pl.loop(_i32(0), _i32(CHUNK))
        def _ones(r):
            for cc in range(cw // LANES):
                ones_v[r, pl.ds(cc * LANES, LANES)] = jnp.ones((LANES,), jnp.float32)

        plsc.subcore_barrier()

        base = wid * _i32(epw)

        @pl.loop(_i32(0), _i32(nchunks))
        def _edges(g):
            off = pl.multiple_of(base + g * _i32(CHUNK), CHUNK)
            pltpu.sync_copy(dst_hbm.at[pl.ds(off, CHUNK)], dst_v)
            pltpu.sync_copy(ones_v, c_sh.at[dst_v], add=True)

        plsc.subcore_barrier()
        pltpu.sync_copy(c_sh.at[pl.ds(row0, rows_per_tile)],
                        out_c.at[cid, pl.ds(row0, rows_per_tile)])

    return pl.kernel(body, out_type=out_type, mesh=mesh, scratch_types=scratch)


def kernel(x, edge_index, edge_attr, params):
    n, d = x.shape
    e = edge_index.shape[1]
    ed = edge_attr.shape[1]
    layers = params['layers']
    h = layers[0]['Wn'].shape[1]
    cw = h

    ec = 64
    gran = 384
    epw = -(-e // (NW * gran)) * gran
    ep = epw * NW
    n_pad = -(-(n + 64) // CHUNK) * CHUNK
    bn_blk = 2000 if n % 2000 == 0 else 1000
    be_blk = 2000 if e % 2000 == 0 else (1000 if e % 1000 == 0 else 500)

    src = edge_index[0].astype(jnp.int32)
    dst = edge_index[1].astype(jnp.int32)
    pad = ep - e
    src_p = jnp.concatenate([src, jnp.zeros((pad,), jnp.int32)])
    dst_pad = n + jnp.arange(pad, dtype=jnp.int32) % jnp.int32(n_pad - n)
    dst_p = jnp.concatenate([dst, dst_pad])
    ea = edge_attr.astype(jnp.float32)

    edge_sc = _make_edge_sc(n_pad, h, epw, epw // ec, ec)
    cnt_sc = _make_cnt_sc(n_pad, epw, epw // CHUNK, cw)

    (cnt2,) = cnt_sc(dst_p)

    xin = x.astype(jnp.float32)
    xprev = None
    skip = params['skip'].astype(jnp.float32)
    for i, p in enumerate(layers):
        wm1a = p['Wm1'][:h]
        wm1b = p['Wm1'][h:]
        xt, pmat = _tc_prep(xin, p['Wn'], p['bn'][None], wm1a, bn_blk)
        q = _tc_qmat(ea, ep, p['We'], p['be'][None], wm1b, p['bm1'][None], be_blk)
        (s2,) = edge_sc(pmat, q, src_p, dst_p)
        if i == 0:
            rs = jnp.zeros((1, 1), jnp.float32)
            xp = xt
        else:
            rs = skip[i - 1].reshape(1, 1).astype(jnp.float32)
            xp = xprev
        out = _tc_post(s2, cnt2, pmat, xt, xp, p['We'], p['be'][None],
                       wm1b, p['bm1'][None], p['Wm2'], p['bm2'][None],
                       p['Wu1'][:h], p['Wu1'][h:], p['bu1'][None],
                       p['Wu2'], p['bu2'][None], p['ln_g'][None],
                       p['ln_b'][None], rs, bn_blk)
        xprev = out
        xin = out
    return xprev


# speedup vs baseline: 2.1923x; 1.9530x over previous
import functools

import jax
import jax.numpy as jnp
from jax import lax
from jax.experimental import pallas as pl
from jax.experimental.pallas import tpu as pltpu
from jax.experimental.pallas import tpu_sc as plsc

NC = 2
NS = 16
NW = NC * NS
CHUNK = 128
LANES = 16


def _i32(v):
    return jnp.int32(v)


def _prep_body(x_ref, wn_ref, bn_ref, wm1a_ref, xt_ref, p_ref):
    xt = jnp.dot(x_ref[...], wn_ref[...], preferred_element_type=jnp.float32)
    xt = xt + bn_ref[...]
    xt_ref[...] = xt
    p_ref[...] = jnp.dot(xt, wm1a_ref[...], preferred_element_type=jnp.float32)


def _tc_prep(x, wn, bn2, wm1a, bn_blk):
    n, d = x.shape
    h = wn.shape[1]
    grid = (n // bn_blk,)
    return pl.pallas_call(
        _prep_body,
        grid=grid,
        in_specs=[
            pl.BlockSpec((bn_blk, d), lambda i: (i, i * 0)),
            pl.BlockSpec((d, h), lambda i: (i * 0, i * 0)),
            pl.BlockSpec((1, h), lambda i: (i * 0, i * 0)),
            pl.BlockSpec((h, h), lambda i: (i * 0, i * 0)),
        ],
        out_specs=[
            pl.BlockSpec((bn_blk, h), lambda i: (i, i * 0)),
            pl.BlockSpec((bn_blk, h), lambda i: (i, i * 0)),
        ],
        out_shape=[
            jax.ShapeDtypeStruct((n, h), jnp.float32),
            jax.ShapeDtypeStruct((n, h), jnp.float32),
        ],
    )(x, wn, bn2, wm1a)


def _qmat_body(ea_ref, we_ref, be_ref, wm1b_ref, bm1_ref, q_ref):
    w2 = jnp.dot(we_ref[...], wm1b_ref[...], preferred_element_type=jnp.float32)
    c = jnp.dot(be_ref[...], wm1b_ref[...], preferred_element_type=jnp.float32)
    c = c + bm1_ref[...]
    q_ref[...] = jnp.dot(ea_ref[...], w2, preferred_element_type=jnp.float32) + c


def _tc_qmat(ea, ep, we, be2, wm1b, bm12, be_blk):
    e, ed = ea.shape
    h = wm1b.shape[1]
    grid = (e // be_blk,)
    return pl.pallas_call(
        _qmat_body,
        grid=grid,
        in_specs=[
            pl.BlockSpec((be_blk, ed), lambda i: (i, i * 0)),
            pl.BlockSpec((ed, h), lambda i: (i * 0, i * 0)),
            pl.BlockSpec((1, h), lambda i: (i * 0, i * 0)),
            pl.BlockSpec((h, h), lambda i: (i * 0, i * 0)),
            pl.BlockSpec((1, h), lambda i: (i * 0, i * 0)),
        ],
        out_specs=pl.BlockSpec((be_blk, h), lambda i: (i, i * 0)),
        out_shape=jax.ShapeDtypeStruct((ep, h), jnp.float32),
    )(ea, we, be2, wm1b, bm12)


def _post_body(s_ref, cnt_ref, p_ref, xt_ref, xp_ref, we_ref, be_ref,
               wm1b_ref, bm1_ref, wm2_ref, bm2_ref, wu1a_ref, wu1b_ref,
               bu1_ref, wu2_ref, bu2_ref, g_ref, b_ref, rs_ref, out_ref):
    f32 = jnp.float32
    c = jnp.dot(be_ref[...], wm1b_ref[...], preferred_element_type=f32)
    c = c + bm1_ref[...]
    qself = jnp.dot(jnp.sum(we_ref[...], axis=0, keepdims=True), wm1b_ref[...],
                    preferred_element_type=f32) + c
    s = s_ref[...]
    big_s = s[0] + s[1] + jnp.maximum(p_ref[...] + qself, 0.0)
    cnt = cnt_ref[...]
    cnt = jnp.maximum(cnt[0, :, 0:1] + cnt[1, :, 0:1] + 1.0, 1.0)
    aggr = jnp.dot(big_s, wm2_ref[...], preferred_element_type=f32) / cnt
    aggr = aggr + bm2_ref[...]
    h2 = jnp.dot(aggr, wu1a_ref[...], preferred_element_type=f32)
    h2 = h2 + jnp.dot(xt_ref[...], wu1b_ref[...], preferred_element_type=f32)
    h2 = jnp.maximum(h2 + bu1_ref[...], 0.0)
    o = jnp.dot(h2, wu2_ref[...], preferred_element_type=f32) + bu2_ref[...]
    mu = jnp.mean(o, axis=-1, keepdims=True)
    var = jnp.mean((o - mu) ** 2, axis=-1, keepdims=True)
    ln = (o - mu) / jnp.sqrt(var + 1e-5) * g_ref[...] + b_ref[...]
    rs = jnp.maximum(rs_ref[0, 0], 0.0)
    out_ref[...] = ln + rs * xp_ref[...]


def _tc_post(s2, cnt2, p, xt, xp, we, be2, wm1b, bm12, wm2, bm22,
             wu1a, wu1b, bu12, wu2, bu22, g2, b2, rs, bn_blk):
    n, h = p.shape
    ed = we.shape[0]
    cw = cnt2.shape[2]
    grid = (n // bn_blk,)
    full = lambda r, cdim: pl.BlockSpec((r, cdim), lambda i: (i * 0, i * 0))
    return pl.pallas_call(
        _post_body,
        grid=grid,
        in_specs=[
            pl.BlockSpec((2, bn_blk, h), lambda i: (i * 0, i, i * 0)),
            pl.BlockSpec((2, bn_blk, cw), lambda i: (i * 0, i, i * 0)),
            pl.BlockSpec((bn_blk, h), lambda i: (i, i * 0)),
            pl.BlockSpec((bn_blk, h), lambda i: (i, i * 0)),
            pl.BlockSpec((bn_blk, h), lambda i: (i, i * 0)),
            full(ed, h), full(1, h), full(h, h), full(1, h),
            full(h, h), full(1, h), full(h, h), full(h, h), full(1, h),
            full(h, h), full(1, h), full(1, h), full(1, h), full(1, 1),
        ],
        out_specs=pl.BlockSpec((bn_blk, h), lambda i: (i, i * 0)),
        out_shape=jax.ShapeDtypeStruct((n, h), jnp.float32),
    )(s2, cnt2, p, xt, xp, we, be2, wm1b, bm12, wm2, bm22,
      wu1a, wu1b, bu12, wu2, bu22, g2, b2, rs)


def _make_edge_sc(n_pad, h, epw, nchunks, ec):
    mesh = plsc.VectorSubcoreMesh(core_axis_name="c", subcore_axis_name="s")
    rows_per_tile = n_pad // NS
    assert nchunks % 3 == 0 and nchunks >= 6
    out_type = [jax.ShapeDtypeStruct((NC, n_pad, h), jnp.float32)]
    scratch = (
        [pltpu.VMEM_SHARED((n_pad, h), jnp.float32)]
        + [pltpu.VMEM((ec,), jnp.int32) for _ in range(3)]
        + [pltpu.VMEM((ec,), jnp.int32) for _ in range(3)]
        + [pltpu.VMEM((ec, h), jnp.float32) for _ in range(3)]
        + [pltpu.VMEM((ec, h), jnp.float32) for _ in range(3)]
        + [pltpu.SemaphoreType.DMA for _ in range(9)]
    )

    def body(p_hbm, q_hbm, src_hbm, dst_hbm, out_s, s_sh, *rest):
        srcv = rest[0:3]
        dstv = rest[3:6]
        rows = rest[6:9]
        qv = rest[9:12]
        gsem = rest[12:15]
        qsem = rest[15:18]
        ssem = rest[18:21]
        cid = lax.axis_index("c").astype(jnp.int32)
        sid = lax.axis_index("s").astype(jnp.int32)
        wid = cid * _i32(NS) + sid
        row0 = sid * _i32(rows_per_tile)
        base = wid * _i32(epw)

        @pl.loop(_i32(0), _i32(ec))
        def _zero_q(r):
            for cc in range(h // LANES):
                qv[0][r, pl.ds(cc * LANES, LANES)] = jnp.zeros((LANES,), jnp.float32)

        left = rows_per_tile
        while left > 0:
            step = min(ec, left)
            pltpu.sync_copy(qv[0].at[pl.ds(0, step)],
                            s_sh.at[pl.ds(row0 + _i32(rows_per_tile - left), step)])
            left -= step

        plsc.subcore_barrier()

        def load(b, g):
            off = pl.multiple_of(base + g * _i32(ec), ec)
            pltpu.sync_copy(src_hbm.at[pl.ds(off, ec)], srcv[b])
            pltpu.sync_copy(dst_hbm.at[pl.ds(off, ec)], dstv[b])
            pltpu.async_copy(p_hbm.at[srcv[b]], rows[b], gsem[b])
            pltpu.async_copy(q_hbm.at[pl.ds(off, ec)], qv[b], qsem[b])

        def process(b):
            pltpu.make_async_copy(p_hbm.at[srcv[b]], rows[b], gsem[b]).wait()
            pltpu.make_async_copy(q_hbm.at[pl.ds(0, ec)], qv[b], qsem[b]).wait()

            @pl.loop(_i32(0), _i32(ec))
            def _relu(r):
                for cc in range(h // LANES):
                    sl = pl.ds(cc * LANES, LANES)
                    qv[b][r, sl] = jnp.maximum(rows[b][r, sl] + qv[b][r, sl], 0.0)

            pltpu.async_copy(qv[b], s_sh.at[dstv[b]], ssem[b], add=True)

        def drain_scatter(b):
            pltpu.make_async_copy(q_hbm.at[pl.ds(0, ec)], qv[b], ssem[b]).wait()

        for b in range(3):
            load(b, _i32(b))

        @pl.loop(_i32(0), _i32(nchunks - 3), step=3)
        def _outer(g0):
            for b in range(3):
                process(b)
            for b in range(3):
                drain_scatter(b)
                load(b, g0 + _i32(b + 3))

        for b in range(3):
            process(b)
        for b in range(3):
            drain_scatter(b)

        plsc.subcore_barrier()
        pltpu.sync_copy(s_sh.at[pl.ds(row0, rows_per_tile)],
                        out_s.at[cid, pl.ds(row0, rows_per_tile)])

    return pl.kernel(body, out_type=out_type, mesh=mesh, scratch_types=scratch)


def _make_cnt_sc(n_pad, epw, nchunks, cw):
    mesh = plsc.VectorSubcoreMesh(core_axis_name="c", subcore_axis_name="s")
    rows_per_tile = n_pad // NS
    out_type = [jax.ShapeDtypeStruct((NC, n_pad, cw), jnp.float32)]
    scratch = [
        pltpu.VMEM_SHARED((n_pad, cw), jnp.float32),
        pltpu.VMEM((CHUNK,), jnp.int32),
        pltpu.VMEM((CHUNK, cw), jnp.float32),
        pltpu.SemaphoreType.DMA,
    ]

    def body(dst_hbm, out_c, c_sh, dst_v, ones_v, sem):
        cid = lax.axis_index("c").astype(jnp.int32)
        sid = lax.axis_index("s").astype(jnp.int32)
        wid = cid * _i32(NS) + sid
        row0 = sid * _i32(rows_per_tile)

        @pl.loop(_i32(0), _i32(CHUNK))
        def _zero(r):
            for cc in range(cw // LANES):
                ones_v[r, pl.ds(cc * LANES, LANES)] = jnp.zeros((LANES,), jnp.float32)

        left = rows_per_tile
        while left > 0:
            step = min(CHUNK, left)
            pltpu.sync_copy(ones_v.at[pl.ds(0, step)],
                            c_sh.at[pl.ds(row0 + _i32(rows_per_tile - left), step)])
            left -= step

        @pl.loop(_i32(0), _i32(CHUNK))
        def _ones(r):
            for cc in range(cw // LANES):
                ones_v[r, pl.ds(cc * LANES, LANES)] = jnp.ones((LANES,), jnp.float32)

        plsc.subcore_barrier()

        base = wid * _i32(epw)

        @pl.loop(_i32(0), _i32(nchunks))
        def _edges(g):
            off = pl.multiple_of(base + g * _i32(CHUNK), CHUNK)
            pltpu.sync_copy(dst_hbm.at[pl.ds(off, CHUNK)], dst_v)
            pltpu.sync_copy(ones_v, c_sh.at[dst_v], add=True)

        plsc.subcore_barrier()
        pltpu.sync_copy(c_sh.at[pl.ds(row0, rows_per_tile)],
                        out_c.at[cid, pl.ds(row0, rows_per_tile)])

    return pl.kernel(body, out_type=out_type, mesh=mesh, scratch_types=scratch)


def kernel(x, edge_index, edge_attr, params):
    n, d = x.shape
    e = edge_index.shape[1]
    ed = edge_attr.shape[1]
    layers = params['layers']
    h = layers[0]['Wn'].shape[1]
    cw = h

    ec = 64
    gran = 384
    epw = -(-e // (NW * gran)) * gran
    ep = epw * NW
    n_pad = -(-(n + 64) // CHUNK) * CHUNK
    bn_blk = 2000 if n % 2000 == 0 else 1000
    be_blk = 2000 if e % 2000 == 0 else (1000 if e % 1000 == 0 else 500)

    src = edge_index[0].astype(jnp.int32)
    dst = edge_index[1].astype(jnp.int32)
    pad = ep - e
    pad_ar = jnp.arange(pad, dtype=jnp.int32)
    src_p = jnp.concatenate([src, pad_ar % jnp.int32(n)])
    dst_pad = n + pad_ar % jnp.int32(n_pad - n)
    dst_p = jnp.concatenate([dst, dst_pad])
    ea = edge_attr.astype(jnp.float32)

    edge_sc = _make_edge_sc(n_pad, h, epw, epw // ec, ec)
    cnt_sc = _make_cnt_sc(n_pad, epw, epw // CHUNK, cw)

    (cnt2,) = cnt_sc(dst_p)

    xin = x.astype(jnp.float32)
    xprev = None
    skip = params['skip'].astype(jnp.float32)
    for i, p in enumerate(layers):
        wm1a = p['Wm1'][:h]
        wm1b = p['Wm1'][h:]
        xt, pmat = _tc_prep(xin, p['Wn'], p['bn'][None], wm1a, bn_blk)
        q = _tc_qmat(ea, ep, p['We'], p['be'][None], wm1b, p['bm1'][None], be_blk)
        (s2,) = edge_sc(pmat, q, src_p, dst_p)
        if i == 0:
            rs = jnp.zeros((1, 1), jnp.float32)
            xp = xt
        else:
            rs = skip[i - 1].reshape(1, 1).astype(jnp.float32)
            xp = xprev
        out = _tc_post(s2, cnt2, pmat, xt, xp, p['We'], p['be'][None],
                       wm1b, p['bm1'][None], p['Wm2'], p['bm2'][None],
                       p['Wu1'][:h], p['Wu1'][h:], p['bu1'][None],
                       p['Wu2'], p['bu2'][None], p['ln_g'][None],
                       p['ln_b'][None], rs, bn_blk)
        xprev = out
        xin = out
    return xprev
